# baseline (speedup 1.0000x reference)
import functools

import jax
import jax.numpy as jnp
from jax import lax
from jax.experimental import pallas as pl
from jax.experimental.pallas import tpu as pltpu
from jax.experimental.pallas import tpu_sc as plsc

NUM_FIELDS = 26
VOCAB = 100000
EMBED_DIM = 32
BATCH = 16384

_info = plsc.get_sparse_core_info()
_NC, _NS, _L = _info.num_cores, _info.num_subcores, _info.num_lanes
NW = _NC * _NS
SW = BATCH // NW
C = 64
NCHUNK = SW // C
RPC = C * NUM_FIELDS
NSTREAM = RPC // 128
IDX_ROWS = SW * NUM_FIELDS // 128
OW = EMBED_DIM * NUM_FIELDS


def _sc_call(tbl, xg):
    mesh = plsc.VectorSubcoreMesh(core_axis_name="c", subcore_axis_name="s")

    @functools.partial(
        pl.kernel,
        mesh=mesh,
        out_type=jax.ShapeDtypeStruct((BATCH * OW,), jnp.float32),
        compiler_params=pltpu.CompilerParams(
            needs_layout_passes=False, use_tc_tiling_on_sc=False),
        scratch_types=[
            pltpu.VMEM((IDX_ROWS, 128), jnp.int32),
            pltpu.VMEM((RPC, EMBED_DIM), jnp.float32),
            pltpu.VMEM((C * OW,), jnp.float32),
            pltpu.SemaphoreType.DMA,
        ],
    )
    def k(tbl_hbm, xg_hbm, out_hbm, idx_v, g_v, o_v, sem):
        wid = lax.axis_index("s") * _NC + lax.axis_index("c")
        pltpu.sync_copy(xg_hbm.at[wid], idx_v)
        lane26 = lax.iota(jnp.int32, _L) * NUM_FIELDS

        for c0 in range(NCHUNK):
            cps = [
                pltpu.async_copy(
                    tbl_hbm.at[idx_v.at[c0 * NSTREAM + j]],
                    g_v.at[pl.ds(j * 128, 128)],
                    sem,
                )
                for j in range(NSTREAM)
            ]
            for cp in cps:
                cp.wait()

            def body(c, carry):
                for f in range(NUM_FIELDS):
                    row = c * NUM_FIELDS + f
                    v0 = g_v[row, pl.ds(0, _L)]
                    v1 = g_v[row, pl.ds(_L, _L)]
                    base = c * OW + f
                    plsc.store_scatter(o_v, [lane26 + base], v0)
                    plsc.store_scatter(
                        o_v, [lane26 + (base + _L * NUM_FIELDS)], v1)
                return carry

            lax.fori_loop(0, C, body, jnp.int32(0))

            off = pl.multiple_of((wid * SW + c0 * C) * OW, OW)
            pltpu.sync_copy(o_v, out_hbm.at[pl.ds(off, C * OW)])

    return k(tbl, xg)


def kernel(x, tables):
    offs = (jnp.arange(NUM_FIELDS, dtype=jnp.int32) * VOCAB)[None, :]
    xg = x.astype(jnp.int32) + offs
    xg = xg.reshape(NW, IDX_ROWS, 128)
    tbl = tables.reshape(NUM_FIELDS * VOCAB, EMBED_DIM)
    out = _sc_call(tbl, xg)
    return out.reshape(BATCH, EMBED_DIM, NUM_FIELDS)


# speedup vs baseline: 1.7030x; 1.7030x over previous
import functools

import jax
import jax.numpy as jnp
from jax import lax
from jax.experimental import pallas as pl
from jax.experimental.pallas import tpu as pltpu
from jax.experimental.pallas import tpu_sc as plsc

NUM_FIELDS = 26
VOCAB = 100000
EMBED_DIM = 32
BATCH = 16384

_info = plsc.get_sparse_core_info()
_NC, _NS, _L = _info.num_cores, _info.num_subcores, _info.num_lanes
NW = _NC * _NS
NPAIRS = NUM_FIELDS * EMBED_DIM
PPW = NPAIRS // NW
SLAB = 2048
NSLAB = BATCH // SLAB


def _sc_call(tabT, xT):
    mesh = plsc.VectorSubcoreMesh(core_axis_name="c", subcore_axis_name="s")

    @functools.partial(
        pl.kernel,
        mesh=mesh,
        out_type=jax.ShapeDtypeStruct((NUM_FIELDS, EMBED_DIM, BATCH),
                                      jnp.float32),
        compiler_params=pltpu.CompilerParams(
            needs_layout_passes=False, use_tc_tiling_on_sc=False),
        scratch_types=[
            pltpu.VMEM((VOCAB,), jnp.float32),
            pltpu.VMEM((SLAB,), jnp.int32),
            pltpu.VMEM((SLAB,), jnp.float32),
            pltpu.SemaphoreType.DMA,
        ],
    )
    def k(tabT_hbm, xT_hbm, out_hbm, row_v, idx_v, o_v, sem):
        wid = lax.axis_index("s") * _NC + lax.axis_index("c")

        def pair_body(p0, carry):
            p = wid * PPW + p0
            f = p // EMBED_DIM
            d = p % EMBED_DIM
            pltpu.async_copy(tabT_hbm.at[f, d], row_v, sem).wait()
            for s in range(NSLAB):
                pltpu.async_copy(
                    xT_hbm.at[f, pl.ds(s * SLAB, SLAB)], idx_v, sem).wait()

                def gather_body(i, c2):
                    idxv = idx_v[pl.ds(i * _L, _L)]
                    o_v[pl.ds(i * _L, _L)] = plsc.load_gather(row_v, [idxv])
                    return c2

                lax.fori_loop(0, SLAB // _L, gather_body, jnp.int32(0))
                pltpu.async_copy(
                    o_v, out_hbm.at[f, d, pl.ds(s * SLAB, SLAB)], sem).wait()
            return carry

        lax.fori_loop(0, PPW, pair_body, jnp.int32(0))

    return k(tabT, xT)


def kernel(x, tables):
    tabT = jnp.transpose(tables, (0, 2, 1))
    xT = x.T
    outT = _sc_call(tabT, xT.astype(jnp.int32))
    return jnp.transpose(outT, (2, 1, 0))


# speedup vs baseline: 3.0748x; 1.8055x over previous
import functools

import jax
import jax.numpy as jnp
from jax import lax
from jax.experimental import pallas as pl
from jax.experimental.pallas import tpu as pltpu
from jax.experimental.pallas import tpu_sc as plsc

NUM_FIELDS = 26
VOCAB = 100000
EMBED_DIM = 32
BATCH = 16384

_info = plsc.get_sparse_core_info()
_NC, _NS, _L = _info.num_cores, _info.num_subcores, _info.num_lanes
NW = _NC * _NS
NPAIRS = NUM_FIELDS * EMBED_DIM
PPW = NPAIRS // NW
SLAB = 2048
NSLAB = BATCH // SLAB


def _sc_call(tabT, xP):
    mesh = plsc.VectorSubcoreMesh(core_axis_name="c", subcore_axis_name="s")

    @functools.partial(
        pl.kernel,
        mesh=mesh,
        out_type=jax.ShapeDtypeStruct((NUM_FIELDS, EMBED_DIM, BATCH),
                                      jnp.float32),
        compiler_params=pltpu.CompilerParams(
            needs_layout_passes=False, use_tc_tiling_on_sc=True),
        scratch_types=[
            pltpu.VMEM((VOCAB,), jnp.float32),
            pltpu.VMEM((SLAB // 128, 128), jnp.int32),
            pltpu.VMEM((SLAB,), jnp.float32),
            pltpu.SemaphoreType.DMA,
        ],
    )
    def k(tabT_hbm, xP_hbm, out_hbm, row_v, idx_v, o_v, sem):
        wid = lax.axis_index("s") * _NC + lax.axis_index("c")

        def pair_body(p0, carry):
            p = wid * PPW + p0
            f = p // EMBED_DIM
            d = p % EMBED_DIM
            pltpu.async_copy(tabT_hbm.at[f, d], row_v, sem).wait()
            for s in range(NSLAB):
                pltpu.async_copy(
                    xP_hbm.at[f, pl.ds(s * (SLAB // 128), SLAB // 128)],
                    idx_v, sem).wait()

                def gather_body(i, c2):
                    idxv = idx_v[i // 8, pl.ds((i % 8) * _L, _L)]
                    o_v[pl.ds(i * _L, _L)] = plsc.load_gather(row_v, [idxv])
                    return c2

                lax.fori_loop(0, SLAB // _L, gather_body, jnp.int32(0))
                pltpu.async_copy(
                    o_v, out_hbm.at[f, d, pl.ds(s * SLAB, SLAB)], sem).wait()
            return carry

        lax.fori_loop(0, PPW, pair_body, jnp.int32(0))

    return k(tabT, xP)


def kernel(x, tables):
    tabT = jnp.transpose(tables, (0, 2, 1))
    xP = x.T.astype(jnp.int32).reshape(NUM_FIELDS, BATCH // 128, 128)
    outT = _sc_call(tabT, xP)
    return jnp.transpose(outT, (2, 1, 0))


# speedup vs baseline: 6.3839x; 2.0762x over previous
import functools

import jax
import jax.numpy as jnp
from jax import lax
from jax.experimental import pallas as pl
from jax.experimental.pallas import tpu as pltpu
from jax.experimental.pallas import tpu_sc as plsc

NUM_FIELDS = 26
VOCAB = 100000
EMBED_DIM = 32
BATCH = 16384

_info = plsc.get_sparse_core_info()
_NC, _NS, _L = _info.num_cores, _info.num_subcores, _info.num_lanes
NW = _NC * _NS
NPAIRS = NUM_FIELDS * EMBED_DIM
PPW = NPAIRS // NW
SLAB = 2048
NSLAB = BATCH // SLAB


def _sc_call(tabT, xP):
    mesh = plsc.VectorSubcoreMesh(core_axis_name="c", subcore_axis_name="s")

    @functools.partial(
        pl.kernel,
        mesh=mesh,
        out_type=jax.ShapeDtypeStruct((NUM_FIELDS, EMBED_DIM, BATCH),
                                      jnp.float32),
        compiler_params=pltpu.CompilerParams(
            needs_layout_passes=False, use_tc_tiling_on_sc=True),
        scratch_types=[
            pltpu.VMEM((VOCAB,), jnp.float32),
            pltpu.VMEM((BATCH // 128, 128), jnp.int32),
            pltpu.VMEM((SLAB,), jnp.float32),
            pltpu.VMEM((SLAB,), jnp.float32),
            pltpu.SemaphoreType.DMA,
            pltpu.SemaphoreType.DMA,
            pltpu.SemaphoreType.DMA,
        ],
    )
    def k(tabT_hbm, xP_hbm, out_hbm, row_v, idx_v, o_v0, o_v1, sem,
          osem0, osem1):
        wid = lax.axis_index("s") * _NC + lax.axis_index("c")
        obufs = (o_v0, o_v1)
        osems = (osem0, osem1)

        def pair_body(p0, prev_f):
            p = wid * PPW + p0
            f = p // EMBED_DIM
            d = p % EMBED_DIM

            @pl.when(f != prev_f)
            def _():
                pltpu.async_copy(xP_hbm.at[f], idx_v, sem).wait()

            pltpu.async_copy(tabT_hbm.at[f, d], row_v, sem).wait()

            for s in range(NSLAB):
                ob = obufs[s % 2]
                if s >= 2:
                    pltpu.make_async_copy(
                        ob, out_hbm.at[f, d, pl.ds((s - 2) * SLAB, SLAB)],
                        osems[s % 2]).wait()

                def gather_body(j, c2):
                    base_row = s * (SLAB // 128) + j
                    for u in range(8):
                        idxv = idx_v[base_row, pl.ds(u * _L, _L)]
                        ob[pl.ds(j * 128 + u * _L, _L)] = (
                            plsc.load_gather(row_v, [idxv]))
                    return c2

                lax.fori_loop(0, SLAB // 128, gather_body, jnp.int32(0))
                pltpu.async_copy(
                    ob, out_hbm.at[f, d, pl.ds(s * SLAB, SLAB)],
                    osems[s % 2])
            for s in (NSLAB - 2, NSLAB - 1):
                pltpu.make_async_copy(
                    obufs[s % 2],
                    out_hbm.at[f, d, pl.ds(s * SLAB, SLAB)],
                    osems[s % 2]).wait()
            return f

        lax.fori_loop(0, PPW, pair_body, jnp.int32(-1))

    return k(tabT, xP)


def kernel(x, tables):
    tabT = jnp.transpose(tables, (0, 2, 1))
    xP = x.T.astype(jnp.int32).reshape(NUM_FIELDS, BATCH // 128, 128)
    outT = _sc_call(tabT, xP)
    return jnp.transpose(outT, (2, 1, 0))
